# Initial kernel scaffold; baseline (speedup 1.0000x reference)
import jax
import jax.numpy as jnp
from jax.experimental import pallas as pl


def kernel(x, edge_index, edge_attr, W_init, fn_W1, fn_b1, fn_W2, fn_b2, att_vec, out_W1, out_b1, bn_gamma, bn_beta, out_W2, out_b2):
    raise NotImplementedError("write your pallas kernel here")


# speedup vs baseline: 5.3814x; 5.3814x over previous
import functools

import jax
import jax.numpy as jnp
from jax import lax
from jax.experimental import pallas as pl
from jax.experimental.pallas import tpu as pltpu
from jax.experimental.pallas import tpu_sc as plsc

N_NODES = 10000
N_EDGES = 320000
D = 128
D_EDGE = 16

NC = 2
NS = 16
NW = NC * NS
CHUNK = 128
NCHUNK = N_EDGES // CHUNK
ROWS_PER_SUB = N_NODES // NS

_PREC = lax.Precision.HIGHEST


def _dot(a, b, dims):
    return lax.dot_general(a, b, (dims, ((), ())), precision=_PREC,
                           preferred_element_type=jnp.float32)


def _k1_body(x_ref, w_ref, h_ref):
    h_ref[...] = _dot(x_ref[...], w_ref[...], ((1,), (1,)))


def _k1(x, w_init):
    return pl.pallas_call(
        _k1_body,
        out_shape=jax.ShapeDtypeStruct((N_NODES, D), jnp.float32),
    )(x, w_init)


def _k2_body(h_hbm, src_hbm, out_hbm, idx_v, row_v, sem):
    wid = lax.axis_index("s") * NC + lax.axis_index("c")
    cnt = NCHUNK // NW + jnp.where(wid < NCHUNK % NW, 1, 0)

    def body(k, _):
        j = wid + k * NW
        eb = j * CHUNK
        pltpu.sync_copy(src_hbm.at[pl.ds(eb, CHUNK)], idx_v)
        pltpu.async_copy(h_hbm.at[idx_v], row_v, sem).wait()
        pltpu.sync_copy(row_v, out_hbm.at[pl.ds(eb, CHUNK)])
        return 0

    lax.fori_loop(0, cnt, body, 0)


def _k2(h, src):
    mesh = plsc.VectorSubcoreMesh(core_axis_name="c", subcore_axis_name="s")
    f = functools.partial(
        pl.kernel,
        out_type=jax.ShapeDtypeStruct((N_EDGES, D), jnp.float32),
        mesh=mesh,
        scratch_types=[
            pltpu.VMEM((CHUNK,), jnp.int32),
            pltpu.VMEM((CHUNK, D), jnp.float32),
            pltpu.SemaphoreType.DMA,
        ],
    )(_k2_body)
    return f(h, src)


EBLK = 2560
NBLK = N_EDGES // EBLK
ABLK = EBLK // 128


def _k3_body(ea_ref, xj_ref, w1_ref, b1_ref, w2_ref, b2_ref, av_ref,
             att_ref, em_ref):
    ea = ea_ref[...]
    nrm = jnp.sqrt(jnp.sum(ea * ea, axis=1, keepdims=True)) + 1e-8
    ean = ea / nrm
    hid = jnp.tanh(_dot(ean, w1_ref[...], ((1,), (1,))) + b1_ref[...])
    ew = _dot(hid, w2_ref[...], ((1,), (1,))) + b2_ref[...]
    m = xj_ref[...] * ew
    att_row = av_ref[...]
    attc = jnp.sum(m * att_row, axis=1, keepdims=True)
    em_ref[...] = m * jnp.exp(attc)
    m3 = m.reshape(ABLK, 128, 128)
    att_ref[...] = jnp.sum(m3 * att_row.reshape(1, 1, 128),
                           axis=2).reshape(1, ABLK, 128)


def _k3(edge_attr, xj, fn_W1, fn_b1, fn_W2, fn_b2, att_vec):
    return pl.pallas_call(
        _k3_body,
        grid=(NBLK,),
        in_specs=[
            pl.BlockSpec((EBLK, D_EDGE), lambda i: (i, 0)),
            pl.BlockSpec((EBLK, D), lambda i: (i, 0)),
            pl.BlockSpec((D, D_EDGE), lambda i: (0, 0)),
            pl.BlockSpec((1, D), lambda i: (0, 0)),
            pl.BlockSpec((D, D), lambda i: (0, 0)),
            pl.BlockSpec((1, D), lambda i: (0, 0)),
            pl.BlockSpec((1, D), lambda i: (0, 0)),
        ],
        out_specs=[
            pl.BlockSpec((1, ABLK, 128), lambda i: (i, 0, 0)),
            pl.BlockSpec((EBLK, D), lambda i: (i, 0)),
        ],
        out_shape=[
            jax.ShapeDtypeStruct((NBLK, ABLK, 128), jnp.float32),
            jax.ShapeDtypeStruct((N_EDGES, D), jnp.float32),
        ],
    )(edge_attr, xj, fn_W1, fn_b1.reshape(1, D), fn_W2, fn_b2.reshape(1, D),
      att_vec.reshape(1, D))


NROWCH = N_NODES // 16


def _k4_body(em_hbm, att_hbm, dst_hbm, zeros_hbm, numer_hbm, denom_hbm,
             idx_v, em_v, att_v, den_v, acc, sem):
    c = lax.axis_index("c")
    s = lax.axis_index("s")
    wid = s * NC + c

    def zb(i, _):
        den_v[pl.ds(i * 16, 16)] = jnp.zeros((16,), jnp.float32)
        return 0
    lax.fori_loop(0, N_NODES // 16, zb, 0)

    zcnt = NROWCH // NS + jnp.where(s < NROWCH % NS, 1, 0)

    def zacc(t, _):
        off = pl.multiple_of(16 * (s + NS * t), 16)
        pltpu.sync_copy(zeros_hbm.at[pl.ds(off, 16)], acc.at[pl.ds(off, 16)])
        return 0
    lax.fori_loop(0, zcnt, zacc, 0)
    plsc.subcore_barrier()

    half = NCHUNK // NC
    cnt = half // NS + jnp.where(s < half % NS, 1, 0)

    def body(k, _):
        j = c * half + s + k * NS
        eb = pl.multiple_of(j * CHUNK, CHUNK)
        pltpu.sync_copy(dst_hbm.at[pl.ds(eb, CHUNK)], idx_v)
        pltpu.sync_copy(em_hbm.at[pl.ds(eb, CHUNK)], em_v)
        pltpu.sync_copy(att_hbm.at[pl.ds(eb, CHUNK)], att_v)
        pltpu.sync_copy(em_v, acc.at[idx_v], add=True)
        for t in range(CHUNK // 16):
            iv = idx_v[pl.ds(t * 16, 16)]
            ev = jnp.exp(att_v[pl.ds(t * 16, 16)])
            plsc.addupdate_scatter(den_v, [iv], ev)
        return 0

    lax.fori_loop(0, cnt, body, 0)
    plsc.subcore_barrier()

    pltpu.sync_copy(den_v, denom_hbm.at[pl.ds(wid * N_NODES, N_NODES)])

    def drain(t, _):
        off = pl.multiple_of(16 * (s + NS * t), 16)
        pltpu.sync_copy(acc.at[pl.ds(off, 16)],
                        numer_hbm.at[c, pl.ds(off, 16)])
        return 0
    lax.fori_loop(0, zcnt, drain, 0)


def _k4(em, att_flat, dst):
    mesh = plsc.VectorSubcoreMesh(core_axis_name="c", subcore_axis_name="s")
    zeros = jnp.zeros((N_NODES, D), jnp.float32)
    f = functools.partial(
        pl.kernel,
        out_type=[
            jax.ShapeDtypeStruct((NC, N_NODES, D), jnp.float32),
            jax.ShapeDtypeStruct((NW * N_NODES,), jnp.float32),
        ],
        mesh=mesh,
        scratch_types=[
            pltpu.VMEM((CHUNK,), jnp.int32),
            pltpu.VMEM((CHUNK, D), jnp.float32),
            pltpu.VMEM((CHUNK,), jnp.float32),
            pltpu.VMEM((N_NODES,), jnp.float32),
            pltpu.VMEM_SHARED((N_NODES, D), jnp.float32),
            pltpu.SemaphoreType.DMA,
        ],
        compiler_params=pltpu.CompilerParams(needs_layout_passes=False),
    )(_k4_body)
    return f(em, att_flat, dst, zeros)


def _k5_body(numer_ref, denom_ref, w1_ref, b1_ref, g_ref, bb_ref, w2_ref,
             b2_ref, out_ref):
    nsum = numer_ref[0] + numer_ref[1]
    dcol = _dot(denom_ref[...], jnp.ones((NW, 1), jnp.float32),
                ((0,), (0,)))
    conv = nsum / (dcol + 1e-16)
    h1 = _dot(conv, w1_ref[...], ((1,), (1,))) + b1_ref[...]
    mean = jnp.mean(h1, axis=0, keepdims=True)
    var = jnp.mean((h1 - mean) ** 2, axis=0, keepdims=True)
    h1 = (h1 - mean) / jnp.sqrt(var + 1e-5) * g_ref[...] + bb_ref[...]
    h1 = jnp.tanh(h1)
    out_ref[...] = _dot(h1, w2_ref[...], ((1,), (1,))) + b2_ref[...]


def _k5(numer, denom, out_W1, out_b1, bn_gamma, bn_beta, out_W2, out_b2):
    return pl.pallas_call(
        _k5_body,
        out_shape=jax.ShapeDtypeStruct((N_NODES, D), jnp.float32),
    )(numer, denom, out_W1, out_b1.reshape(1, D), bn_gamma.reshape(1, D),
      bn_beta.reshape(1, D), out_W2, out_b2.reshape(1, D))


def kernel(x, edge_index, edge_attr, W_init, fn_W1, fn_b1, fn_W2, fn_b2,
           att_vec, out_W1, out_b1, bn_gamma, bn_beta, out_W2, out_b2):
    src = edge_index[0]
    dst = edge_index[1]
    h = _k1(x, W_init)
    xj = _k2(h, src)
    att2, em = _k3(edge_attr, xj, fn_W1, fn_b1, fn_W2, fn_b2, att_vec)
    att = att2.reshape(N_EDGES)
    numer, denom = _k4(em, att, dst)
    out = _k5(numer, denom.reshape(NW, N_NODES), out_W1, out_b1, bn_gamma,
              bn_beta, out_W2, out_b2)
    return out, att


# speedup vs baseline: 7.9379x; 1.4751x over previous
import functools

import jax
import jax.numpy as jnp
from jax import lax
from jax.experimental import pallas as pl
from jax.experimental.pallas import tpu as pltpu
from jax.experimental.pallas import tpu_sc as plsc

N_NODES = 10000
N_EDGES = 320000
D = 128
D_EDGE = 16

NC = 2
NS = 16
NW = NC * NS
CHUNK = 128
NCHUNK = N_EDGES // CHUNK
ROWS_PER_SUB = N_NODES // NS

_PREC = lax.Precision.DEFAULT


def _dot(a, b, dims):
    return lax.dot_general(a, b, (dims, ((), ())), precision=_PREC,
                           preferred_element_type=jnp.float32)


def _k1_body(x_ref, w_ref, h_ref):
    h_ref[...] = _dot(x_ref[...], w_ref[...], ((1,), (1,)))


def _k1(x, w_init):
    return pl.pallas_call(
        _k1_body,
        out_shape=jax.ShapeDtypeStruct((N_NODES, D), jnp.float32),
    )(x, w_init)


def _k2_body(h_hbm, src_hbm, out_hbm, idx_v, row_v, sem):
    wid = lax.axis_index("s") * NC + lax.axis_index("c")
    cnt = NCHUNK // NW + jnp.where(wid < NCHUNK % NW, 1, 0)

    def body(k, _):
        j = wid + k * NW
        eb = j * CHUNK
        pltpu.sync_copy(src_hbm.at[pl.ds(eb, CHUNK)], idx_v)
        pltpu.async_copy(h_hbm.at[idx_v], row_v, sem).wait()
        pltpu.sync_copy(row_v, out_hbm.at[pl.ds(eb, CHUNK)])
        return 0

    lax.fori_loop(0, cnt, body, 0)


def _k2(h, src):
    mesh = plsc.VectorSubcoreMesh(core_axis_name="c", subcore_axis_name="s")
    f = functools.partial(
        pl.kernel,
        out_type=jax.ShapeDtypeStruct((N_EDGES, D), jnp.float32),
        mesh=mesh,
        scratch_types=[
            pltpu.VMEM((CHUNK,), jnp.int32),
            pltpu.VMEM((CHUNK, D), jnp.float32),
            pltpu.SemaphoreType.DMA,
        ],
    )(_k2_body)
    return f(h, src)


EBLK = 2560
NBLK = N_EDGES // EBLK
ABLK = EBLK // 128


def _k3_body(ea_ref, xj_ref, w1_ref, b1_ref, w2_ref, b2_ref, av_ref,
             att_ref, em_ref):
    ea = ea_ref[...]
    nrm = jnp.sqrt(jnp.sum(ea * ea, axis=1, keepdims=True)) + 1e-8
    ean = ea / nrm
    hid = jnp.tanh(_dot(ean, w1_ref[...], ((1,), (1,))) + b1_ref[...])
    ew = _dot(hid, w2_ref[...], ((1,), (1,))) + b2_ref[...]
    m = xj_ref[...] * ew
    att_row = av_ref[...]
    attc = jnp.sum(m * att_row, axis=1, keepdims=True)
    em_ref[...] = m * jnp.exp(attc)
    m3 = m.reshape(ABLK, 128, 128)
    att_ref[...] = jnp.sum(m3 * att_row.reshape(1, 1, 128),
                           axis=2).reshape(1, ABLK, 128)


def _k3(edge_attr, xj, fn_W1, fn_b1, fn_W2, fn_b2, att_vec):
    return pl.pallas_call(
        _k3_body,
        grid=(NBLK,),
        in_specs=[
            pl.BlockSpec((EBLK, D_EDGE), lambda i: (i, 0)),
            pl.BlockSpec((EBLK, D), lambda i: (i, 0)),
            pl.BlockSpec((D, D_EDGE), lambda i: (0, 0)),
            pl.BlockSpec((1, D), lambda i: (0, 0)),
            pl.BlockSpec((D, D), lambda i: (0, 0)),
            pl.BlockSpec((1, D), lambda i: (0, 0)),
            pl.BlockSpec((1, D), lambda i: (0, 0)),
        ],
        out_specs=[
            pl.BlockSpec((1, ABLK, 128), lambda i: (i, 0, 0)),
            pl.BlockSpec((EBLK, D), lambda i: (i, 0)),
        ],
        out_shape=[
            jax.ShapeDtypeStruct((NBLK, ABLK, 128), jnp.float32),
            jax.ShapeDtypeStruct((N_EDGES, D), jnp.float32),
        ],
    )(edge_attr, xj, fn_W1, fn_b1.reshape(1, D), fn_W2, fn_b2.reshape(1, D),
      att_vec.reshape(1, D))


NROWCH = N_NODES // 16


def _k4_body(em_hbm, att_hbm, dst_hbm, zeros_hbm, numer_hbm, denom_hbm,
             idx_v, em_v, att_v, den_v, acc, sem):
    c = lax.axis_index("c")
    s = lax.axis_index("s")
    wid = s * NC + c

    def zb(i, _):
        den_v[pl.ds(i * 16, 16)] = jnp.zeros((16,), jnp.float32)
        return 0
    lax.fori_loop(0, N_NODES // 16, zb, 0)

    zcnt = NROWCH // NS + jnp.where(s < NROWCH % NS, 1, 0)

    def zacc(t, _):
        off = pl.multiple_of(16 * (s + NS * t), 16)
        pltpu.sync_copy(zeros_hbm.at[pl.ds(off, 16)], acc.at[pl.ds(off, 16)])
        return 0
    lax.fori_loop(0, zcnt, zacc, 0)
    plsc.subcore_barrier()

    half = NCHUNK // NC
    cnt = half // NS + jnp.where(s < half % NS, 1, 0)

    def body(k, _):
        j = c * half + s + k * NS
        eb = pl.multiple_of(j * CHUNK, CHUNK)
        pltpu.sync_copy(dst_hbm.at[pl.ds(eb, CHUNK)], idx_v)
        pltpu.sync_copy(em_hbm.at[pl.ds(eb, CHUNK)], em_v)
        pltpu.sync_copy(att_hbm.at[pl.ds(eb, CHUNK)], att_v)
        pltpu.sync_copy(em_v, acc.at[idx_v], add=True)
        for t in range(CHUNK // 16):
            iv = idx_v[pl.ds(t * 16, 16)]
            ev = jnp.exp(att_v[pl.ds(t * 16, 16)])
            plsc.addupdate_scatter(den_v, [iv], ev)
        return 0

    lax.fori_loop(0, cnt, body, 0)
    plsc.subcore_barrier()

    pltpu.sync_copy(den_v, denom_hbm.at[pl.ds(wid * N_NODES, N_NODES)])

    def drain(t, _):
        off = pl.multiple_of(16 * (s + NS * t), 16)
        pltpu.sync_copy(acc.at[pl.ds(off, 16)],
                        numer_hbm.at[c, pl.ds(off, 16)])
        return 0
    lax.fori_loop(0, zcnt, drain, 0)


def _k4(em, att_flat, dst):
    mesh = plsc.VectorSubcoreMesh(core_axis_name="c", subcore_axis_name="s")
    zeros = jnp.zeros((N_NODES, D), jnp.float32)
    f = functools.partial(
        pl.kernel,
        out_type=[
            jax.ShapeDtypeStruct((NC, N_NODES, D), jnp.float32),
            jax.ShapeDtypeStruct((NW * N_NODES,), jnp.float32),
        ],
        mesh=mesh,
        scratch_types=[
            pltpu.VMEM((CHUNK,), jnp.int32),
            pltpu.VMEM((CHUNK, D), jnp.float32),
            pltpu.VMEM((CHUNK,), jnp.float32),
            pltpu.VMEM((N_NODES,), jnp.float32),
            pltpu.VMEM_SHARED((N_NODES, D), jnp.float32),
            pltpu.SemaphoreType.DMA,
        ],
        compiler_params=pltpu.CompilerParams(needs_layout_passes=False),
    )(_k4_body)
    return f(em, att_flat, dst, zeros)


def _k5_body(numer_ref, denom_ref, w1_ref, b1_ref, g_ref, bb_ref, w2_ref,
             b2_ref, out_ref):
    nsum = numer_ref[0] + numer_ref[1]
    dcol = _dot(denom_ref[...], jnp.ones((NW, 1), jnp.float32),
                ((0,), (0,)))
    conv = nsum / (dcol + 1e-16)
    h1 = _dot(conv, w1_ref[...], ((1,), (1,))) + b1_ref[...]
    mean = jnp.mean(h1, axis=0, keepdims=True)
    var = jnp.mean((h1 - mean) ** 2, axis=0, keepdims=True)
    h1 = (h1 - mean) / jnp.sqrt(var + 1e-5) * g_ref[...] + bb_ref[...]
    h1 = jnp.tanh(h1)
    out_ref[...] = _dot(h1, w2_ref[...], ((1,), (1,))) + b2_ref[...]


def _k5(numer, denom, out_W1, out_b1, bn_gamma, bn_beta, out_W2, out_b2):
    return pl.pallas_call(
        _k5_body,
        out_shape=jax.ShapeDtypeStruct((N_NODES, D), jnp.float32),
    )(numer, denom, out_W1, out_b1.reshape(1, D), bn_gamma.reshape(1, D),
      bn_beta.reshape(1, D), out_W2, out_b2.reshape(1, D))


def kernel(x, edge_index, edge_attr, W_init, fn_W1, fn_b1, fn_W2, fn_b2,
           att_vec, out_W1, out_b1, bn_gamma, bn_beta, out_W2, out_b2):
    src = edge_index[0]
    dst = edge_index[1]
    h = _k1(x, W_init)
    xj = _k2(h, src)
    att2, em = _k3(edge_attr, xj, fn_W1, fn_b1, fn_W2, fn_b2, att_vec)
    att = att2.reshape(N_EDGES)
    numer, denom = _k4(em, att, dst)
    out = _k5(numer, denom.reshape(NW, N_NODES), out_W1, out_b1, bn_gamma,
              bn_beta, out_W2, out_b2)
    return out, att


# speedup vs baseline: 9.8517x; 1.2411x over previous
import functools

import jax
import jax.numpy as jnp
from jax import lax
from jax.experimental import pallas as pl
from jax.experimental.pallas import tpu as pltpu
from jax.experimental.pallas import tpu_sc as plsc

N_NODES = 10000
N_EDGES = 320000
D = 128
D_EDGE = 16

NC = 2
NS = 16
NW = NC * NS
CHUNK = 128
NCHUNK = N_EDGES // CHUNK
ROWS_PER_SUB = N_NODES // NS

_PREC = lax.Precision.DEFAULT


def _dot(a, b, dims):
    return lax.dot_general(a, b, (dims, ((), ())), precision=_PREC,
                           preferred_element_type=jnp.float32)


def _k1_body(x_ref, w_ref, h_ref):
    h_ref[...] = _dot(x_ref[...], w_ref[...], ((1,), (1,)))


def _k1(x, w_init):
    return pl.pallas_call(
        _k1_body,
        out_shape=jax.ShapeDtypeStruct((N_NODES, D), jnp.float32),
    )(x, w_init)


CNT2 = NCHUNK // NW
REM2 = NCHUNK - CNT2 * NW
NBUF = 3
NGRP = CNT2 // NBUF


def _k2_body(h_hbm, src_hbm, out_hbm, idx_v, row_v,
             sg0, sg1, sg2, ss0, ss1, ss2):
    wid = lax.axis_index("s") * NC + lax.axis_index("c")
    base = wid * (CNT2 * CHUNK)
    semg = [sg0, sg1, sg2]
    sems = [ss0, ss1, ss2]

    pltpu.sync_copy(src_hbm.at[pl.ds(base, CNT2 * CHUNK)], idx_v)

    def group(g, _):
        ds = []
        for b in range(NBUF):
            off = pl.multiple_of((g * NBUF + b) * CHUNK, CHUNK)
            ds.append(pltpu.async_copy(
                h_hbm.at[idx_v.at[pl.ds(off, CHUNK)]], row_v.at[b], semg[b]))
        ss = []
        for b in range(NBUF):
            off = pl.multiple_of((g * NBUF + b) * CHUNK, CHUNK)
            ds[b].wait()
            ss.append(pltpu.async_copy(
                row_v.at[b], out_hbm.at[pl.ds(base + off, CHUNK)], sems[b]))
        for b in range(NBUF):
            ss[b].wait()
        return 0

    lax.fori_loop(0, NGRP, group, 0)

    @pl.when(wid < REM2)
    def _():
        eb = pl.multiple_of((CNT2 * NW + wid) * CHUNK, CHUNK)
        pltpu.sync_copy(src_hbm.at[pl.ds(eb, CHUNK)],
                        idx_v.at[pl.ds(0, CHUNK)])
        pltpu.async_copy(h_hbm.at[idx_v.at[pl.ds(0, CHUNK)]], row_v.at[0],
                         sg0).wait()
        pltpu.sync_copy(row_v.at[0], out_hbm.at[pl.ds(eb, CHUNK)])


def _k2(h, src):
    mesh = plsc.VectorSubcoreMesh(core_axis_name="c", subcore_axis_name="s")
    f = functools.partial(
        pl.kernel,
        out_type=jax.ShapeDtypeStruct((N_EDGES, D), jnp.float32),
        mesh=mesh,
        scratch_types=[
            pltpu.VMEM((CNT2 * CHUNK,), jnp.int32),
            pltpu.VMEM((NBUF, CHUNK, D), jnp.float32),
            pltpu.SemaphoreType.DMA,
            pltpu.SemaphoreType.DMA,
            pltpu.SemaphoreType.DMA,
            pltpu.SemaphoreType.DMA,
            pltpu.SemaphoreType.DMA,
            pltpu.SemaphoreType.DMA,
        ],
    )(_k2_body)
    return f(h, src)


EBLK = 2560
NBLK = N_EDGES // EBLK
ABLK = EBLK // 128


def _k3_body(ea_ref, xj_ref, w1_ref, b1_ref, w2_ref, b2_ref, av_ref,
             att_ref, em_ref):
    ea = ea_ref[...]
    nrm = jnp.sqrt(jnp.sum(ea * ea, axis=1, keepdims=True)) + 1e-8
    ean = ea / nrm
    hid = jnp.tanh(_dot(ean, w1_ref[...], ((1,), (1,))) + b1_ref[...])
    ew = _dot(hid, w2_ref[...], ((1,), (1,))) + b2_ref[...]
    m = xj_ref[...] * ew
    att_row = av_ref[...]
    attc = jnp.sum(m * att_row, axis=1, keepdims=True)
    em_ref[...] = m * jnp.exp(attc)
    m3 = m.reshape(ABLK, 128, 128)
    att_ref[...] = jnp.sum(m3 * att_row.reshape(1, 1, 128),
                           axis=2).reshape(1, ABLK, 128)


def _k3(edge_attr, xj, fn_W1, fn_b1, fn_W2, fn_b2, att_vec):
    return pl.pallas_call(
        _k3_body,
        grid=(NBLK,),
        in_specs=[
            pl.BlockSpec((EBLK, D_EDGE), lambda i: (i, 0)),
            pl.BlockSpec((EBLK, D), lambda i: (i, 0)),
            pl.BlockSpec((D, D_EDGE), lambda i: (0, 0)),
            pl.BlockSpec((1, D), lambda i: (0, 0)),
            pl.BlockSpec((D, D), lambda i: (0, 0)),
            pl.BlockSpec((1, D), lambda i: (0, 0)),
            pl.BlockSpec((1, D), lambda i: (0, 0)),
        ],
        out_specs=[
            pl.BlockSpec((1, ABLK, 128), lambda i: (i, 0, 0)),
            pl.BlockSpec((EBLK, D), lambda i: (i, 0)),
        ],
        out_shape=[
            jax.ShapeDtypeStruct((NBLK, ABLK, 128), jnp.float32),
            jax.ShapeDtypeStruct((N_EDGES, D), jnp.float32),
        ],
    )(edge_attr, xj, fn_W1, fn_b1.reshape(1, D), fn_W2, fn_b2.reshape(1, D),
      att_vec.reshape(1, D))


NROWCH = N_NODES // 16
CNT4 = (NCHUNK // NC) // NS
NBUF4 = 2


def _k4_body(em_hbm, att_hbm, dst_hbm, zeros_hbm, numer_hbm, denom_hbm,
             idx2, em_v, att_v, den_v, acc, sl0, sl1, sc0, sc1):
    c = lax.axis_index("c")
    s = lax.axis_index("s")
    wid = s * NC + c

    def zb(i, _):
        den_v[pl.ds(i * 16, 16)] = jnp.zeros((16,), jnp.float32)
        return 0
    lax.fori_loop(0, N_NODES // 16, zb, 0)

    zcnt = NROWCH // NS + jnp.where(s < NROWCH % NS, 1, 0)

    def zacc(t, _):
        off = pl.multiple_of(16 * (s + NS * t), 16)
        pltpu.sync_copy(zeros_hbm.at[pl.ds(off, 16)], acc.at[pl.ds(off, 16)])
        return 0
    lax.fori_loop(0, zcnt, zacc, 0)
    plsc.subcore_barrier()

    half = NCHUNK // NC
    start = c * half + s * CNT4
    base = start * CHUNK
    seml = [sl0, sl1]
    semc = [sc0, sc1]

    def denom_chunk(b):
        for t in range(CHUNK // 16):
            iv = idx2[b, pl.ds(t * 16, 16)]
            ev = jnp.exp(att_v[b, pl.ds(t * 16, 16)])
            plsc.addupdate_scatter(den_v, [iv], ev)

    def group(g, _):
        ls = []
        for b in range(NBUF4):
            off = pl.multiple_of(base + (g * NBUF4 + b) * CHUNK, CHUNK)
            ls.append((
                pltpu.async_copy(dst_hbm.at[pl.ds(off, CHUNK)], idx2.at[b],
                                 seml[b]),
                pltpu.async_copy(em_hbm.at[pl.ds(off, CHUNK)], em_v.at[b],
                                 seml[b]),
                pltpu.async_copy(att_hbm.at[pl.ds(off, CHUNK)], att_v.at[b],
                                 seml[b]),
            ))
        cs = []
        for b in range(NBUF4):
            for d in ls[b]:
                d.wait()
            cs.append(pltpu.async_copy(em_v.at[b], acc.at[idx2.at[b]],
                                       semc[b], add=True))
            denom_chunk(b)
        for d in cs:
            d.wait()
        return 0

    lax.fori_loop(0, CNT4 // NBUF4, group, 0)

    @pl.when(s < half - NS * CNT4)
    def _():
        off = pl.multiple_of((c * half + NS * CNT4 + s) * CHUNK, CHUNK)
        pltpu.sync_copy(dst_hbm.at[pl.ds(off, CHUNK)], idx2.at[0])
        pltpu.sync_copy(em_hbm.at[pl.ds(off, CHUNK)], em_v.at[0])
        pltpu.sync_copy(att_hbm.at[pl.ds(off, CHUNK)], att_v.at[0])
        pltpu.sync_copy(em_v.at[0], acc.at[idx2.at[0]], add=True)
        denom_chunk(0)

    plsc.subcore_barrier()

    pltpu.sync_copy(den_v, denom_hbm.at[pl.ds(wid * N_NODES, N_NODES)])

    def drain(t, _):
        off = pl.multiple_of(16 * (s + NS * t), 16)
        pltpu.sync_copy(acc.at[pl.ds(off, 16)],
                        numer_hbm.at[c, pl.ds(off, 16)])
        return 0
    lax.fori_loop(0, zcnt, drain, 0)


def _k4(em, att_flat, dst):
    mesh = plsc.VectorSubcoreMesh(core_axis_name="c", subcore_axis_name="s")
    zeros = jnp.zeros((N_NODES, D), jnp.float32)
    f = functools.partial(
        pl.kernel,
        out_type=[
            jax.ShapeDtypeStruct((NC, N_NODES, D), jnp.float32),
            jax.ShapeDtypeStruct((NW * N_NODES,), jnp.float32),
        ],
        mesh=mesh,
        scratch_types=[
            pltpu.VMEM((NBUF4, CHUNK), jnp.int32),
            pltpu.VMEM((NBUF4, CHUNK, D), jnp.float32),
            pltpu.VMEM((NBUF4, CHUNK), jnp.float32),
            pltpu.VMEM((N_NODES,), jnp.float32),
            pltpu.VMEM_SHARED((N_NODES, D), jnp.float32),
            pltpu.SemaphoreType.DMA,
            pltpu.SemaphoreType.DMA,
            pltpu.SemaphoreType.DMA,
            pltpu.SemaphoreType.DMA,
        ],
        compiler_params=pltpu.CompilerParams(needs_layout_passes=False),
    )(_k4_body)
    return f(em, att_flat, dst, zeros)


def _k5_body(numer_ref, denom_ref, w1_ref, b1_ref, g_ref, bb_ref, w2_ref,
             b2_ref, out_ref):
    nsum = numer_ref[0] + numer_ref[1]
    dcol = _dot(denom_ref[...], jnp.ones((NW, 1), jnp.float32),
                ((0,), (0,)))
    conv = nsum / (dcol + 1e-16)
    h1 = _dot(conv, w1_ref[...], ((1,), (1,))) + b1_ref[...]
    mean = jnp.mean(h1, axis=0, keepdims=True)
    var = jnp.mean((h1 - mean) ** 2, axis=0, keepdims=True)
    h1 = (h1 - mean) / jnp.sqrt(var + 1e-5) * g_ref[...] + bb_ref[...]
    h1 = jnp.tanh(h1)
    out_ref[...] = _dot(h1, w2_ref[...], ((1,), (1,))) + b2_ref[...]


def _k5(numer, denom, out_W1, out_b1, bn_gamma, bn_beta, out_W2, out_b2):
    return pl.pallas_call(
        _k5_body,
        out_shape=jax.ShapeDtypeStruct((N_NODES, D), jnp.float32),
    )(numer, denom, out_W1, out_b1.reshape(1, D), bn_gamma.reshape(1, D),
      bn_beta.reshape(1, D), out_W2, out_b2.reshape(1, D))


def kernel(x, edge_index, edge_attr, W_init, fn_W1, fn_b1, fn_W2, fn_b2,
           att_vec, out_W1, out_b1, bn_gamma, bn_beta, out_W2, out_b2):
    src = edge_index[0]
    dst = edge_index[1]
    h = _k1(x, W_init)
    xj = _k2(h, src)
    att2, em = _k3(edge_attr, xj, fn_W1, fn_b1, fn_W2, fn_b2, att_vec)
    att = att2.reshape(N_EDGES)
    numer, denom = _k4(em, att, dst)
    out = _k5(numer, denom.reshape(NW, N_NODES), out_W1, out_b1, bn_gamma,
              bn_beta, out_W2, out_b2)
    return out, att


# speedup vs baseline: 10.2349x; 1.0389x over previous
import functools

import jax
import jax.numpy as jnp
from jax import lax
from jax.experimental import pallas as pl
from jax.experimental.pallas import tpu as pltpu
from jax.experimental.pallas import tpu_sc as plsc

N_NODES = 10000
N_EDGES = 320000
D = 128
D_EDGE = 16

NC = 2
NS = 16
NW = NC * NS
CHUNK = 128
NSLICE = 2
SLICE_E = N_EDGES // NSLICE
SLICE_CH = SLICE_E // CHUNK

_PREC = lax.Precision.DEFAULT


def _dot(a, b, dims):
    return lax.dot_general(a, b, (dims, ((), ())), precision=_PREC,
                           preferred_element_type=jnp.float32)


def _k1_body(x_ref, w_ref, h_ref):
    h_ref[...] = _dot(x_ref[...], w_ref[...], ((1,), (1,)))


def _k1(x, w_init):
    return pl.pallas_call(
        _k1_body,
        out_shape=jax.ShapeDtypeStruct((N_NODES, D), jnp.float32),
    )(x, w_init)


CNT2 = SLICE_CH // NW
REM2 = SLICE_CH - CNT2 * NW
NBUF = 3
NGRP = CNT2 // NBUF


def _k2_body(e0, h_hbm, src_hbm, out_hbm, idx_v, row_v,
             sg0, sg1, sg2, ss0, ss1, ss2):
    wid = lax.axis_index("s") * NC + lax.axis_index("c")
    base = wid * (CNT2 * CHUNK)
    semg = [sg0, sg1, sg2]
    sems = [ss0, ss1, ss2]

    pltpu.sync_copy(src_hbm.at[pl.ds(e0 + base, CNT2 * CHUNK)], idx_v)

    def group(g, _):
        ds = []
        for b in range(NBUF):
            off = pl.multiple_of((g * NBUF + b) * CHUNK, CHUNK)
            ds.append(pltpu.async_copy(
                h_hbm.at[idx_v.at[pl.ds(off, CHUNK)]], row_v.at[b], semg[b]))
        ss = []
        for b in range(NBUF):
            off = pl.multiple_of((g * NBUF + b) * CHUNK, CHUNK)
            ds[b].wait()
            ss.append(pltpu.async_copy(
                row_v.at[b], out_hbm.at[pl.ds(base + off, CHUNK)], sems[b]))
        for b in range(NBUF):
            ss[b].wait()
        return 0

    lax.fori_loop(0, NGRP, group, 0)

    @pl.when(wid < REM2)
    def _():
        off = pl.multiple_of((CNT2 * NW + wid) * CHUNK, CHUNK)
        pltpu.sync_copy(src_hbm.at[pl.ds(e0 + off, CHUNK)],
                        idx_v.at[pl.ds(0, CHUNK)])
        pltpu.async_copy(h_hbm.at[idx_v.at[pl.ds(0, CHUNK)]], row_v.at[0],
                         sg0).wait()
        pltpu.sync_copy(row_v.at[0], out_hbm.at[pl.ds(off, CHUNK)])


def _k2(h, src, e0):
    mesh = plsc.VectorSubcoreMesh(core_axis_name="c", subcore_axis_name="s")
    f = functools.partial(
        pl.kernel,
        out_type=jax.ShapeDtypeStruct((SLICE_E, D), jnp.float32),
        mesh=mesh,
        scratch_types=[
            pltpu.VMEM((CNT2 * CHUNK,), jnp.int32),
            pltpu.VMEM((NBUF, CHUNK, D), jnp.float32),
            pltpu.SemaphoreType.DMA,
            pltpu.SemaphoreType.DMA,
            pltpu.SemaphoreType.DMA,
            pltpu.SemaphoreType.DMA,
            pltpu.SemaphoreType.DMA,
            pltpu.SemaphoreType.DMA,
        ],
    )(functools.partial(_k2_body, e0))
    return f(h, src)


EBLK = 3200
NBLK = SLICE_E // EBLK
ABLK = EBLK // 128


def _k3_body(ea_ref, xj_ref, w1_ref, b1_ref, w2_ref, b2_ref, av_ref,
             att_ref, em_ref):
    ea = ea_ref[...]
    nrm = jnp.sqrt(jnp.sum(ea * ea, axis=1, keepdims=True)) + 1e-8
    ean = ea / nrm
    hid = jnp.tanh(_dot(ean, w1_ref[...], ((1,), (1,))) + b1_ref[...])
    ew = _dot(hid, w2_ref[...], ((1,), (1,))) + b2_ref[...]
    m = xj_ref[...] * ew
    att_row = av_ref[...]
    attc = jnp.sum(m * att_row, axis=1, keepdims=True)
    em_ref[...] = m * jnp.exp(attc)
    m3 = m.reshape(ABLK, 128, 128)
    att_ref[...] = jnp.sum(m3 * att_row.reshape(1, 1, 128),
                           axis=2).reshape(1, ABLK, 128)


def _k3(edge_attr, xj, fn_W1, fn_b1, fn_W2, fn_b2, att_vec, sl):
    blk0 = sl * NBLK
    return pl.pallas_call(
        _k3_body,
        grid=(NBLK,),
        in_specs=[
            pl.BlockSpec((EBLK, D_EDGE), lambda i: (i + blk0, 0)),
            pl.BlockSpec((EBLK, D), lambda i: (i, 0)),
            pl.BlockSpec((D, D_EDGE), lambda i: (0, 0)),
            pl.BlockSpec((1, D), lambda i: (0, 0)),
            pl.BlockSpec((D, D), lambda i: (0, 0)),
            pl.BlockSpec((1, D), lambda i: (0, 0)),
            pl.BlockSpec((1, D), lambda i: (0, 0)),
        ],
        out_specs=[
            pl.BlockSpec((1, ABLK, 128), lambda i: (i, 0, 0)),
            pl.BlockSpec((EBLK, D), lambda i: (i, 0)),
        ],
        out_shape=[
            jax.ShapeDtypeStruct((NBLK, ABLK, 128), jnp.float32),
            jax.ShapeDtypeStruct((SLICE_E, D), jnp.float32),
        ],
    )(edge_attr, xj, fn_W1, fn_b1.reshape(1, D), fn_W2, fn_b2.reshape(1, D),
      att_vec.reshape(1, D))


CNT4 = (SLICE_CH // NC) // NS
REM4 = SLICE_CH // NC - CNT4 * NS
NBUF4 = 2
NGRP4 = CNT4 // NBUF4


def _k4_body(e0, em_hbm, att_hbm, dst_hbm, zeros_hbm, numer_hbm, denom_hbm,
             idx2, em_v, att_v, den_v, acc, sl0, sl1, sc0, sc1):
    c = lax.axis_index("c")
    s = lax.axis_index("s")
    wid = s * NC + c

    def zb(i, _):
        den_v[pl.ds(i * 16, 16)] = jnp.zeros((16,), jnp.float32)
        return 0
    lax.fori_loop(0, N_NODES // 16, zb, 0)

    nrowch = N_NODES // 16
    zcnt = nrowch // NS + jnp.where(s < nrowch % NS, 1, 0)

    def zacc(t, _):
        off = pl.multiple_of(16 * (s + NS * t), 16)
        pltpu.sync_copy(zeros_hbm.at[pl.ds(off, 16)], acc.at[pl.ds(off, 16)])
        return 0
    lax.fori_loop(0, zcnt, zacc, 0)
    plsc.subcore_barrier()

    half = SLICE_CH // NC
    start = c * half + s * CNT4
    base = start * CHUNK
    seml = [sl0, sl1]
    semc = [sc0, sc1]

    def denom_chunk(b):
        for t in range(CHUNK // 16):
            iv = idx2[b, pl.ds(t * 16, 16)]
            ev = jnp.exp(att_v[b, pl.ds(t * 16, 16)])
            plsc.addupdate_scatter(den_v, [iv], ev)

    def do_chunk_sync(off):
        pltpu.sync_copy(dst_hbm.at[pl.ds(e0 + off, CHUNK)], idx2.at[0])
        pltpu.sync_copy(em_hbm.at[pl.ds(off, CHUNK)], em_v.at[0])
        pltpu.sync_copy(att_hbm.at[pl.ds(off, CHUNK)], att_v.at[0])
        pltpu.sync_copy(em_v.at[0], acc.at[idx2.at[0]], add=True)
        denom_chunk(0)

    def group(g, _):
        ls = []
        for b in range(NBUF4):
            off = pl.multiple_of(base + (g * NBUF4 + b) * CHUNK, CHUNK)
            ls.append((
                pltpu.async_copy(dst_hbm.at[pl.ds(e0 + off, CHUNK)],
                                 idx2.at[b], seml[b]),
                pltpu.async_copy(em_hbm.at[pl.ds(off, CHUNK)], em_v.at[b],
                                 seml[b]),
                pltpu.async_copy(att_hbm.at[pl.ds(off, CHUNK)], att_v.at[b],
                                 seml[b]),
            ))
        cs = []
        for b in range(NBUF4):
            for d in ls[b]:
                d.wait()
            cs.append(pltpu.async_copy(em_v.at[b], acc.at[idx2.at[b]],
                                       semc[b], add=True))
            denom_chunk(b)
        for d in cs:
            d.wait()
        return 0

    lax.fori_loop(0, NGRP4, group, 0)
    do_chunk_sync(pl.multiple_of(base + NGRP4 * NBUF4 * CHUNK, CHUNK))
    @pl.when(s < REM4)
    def _():
        do_chunk_sync(pl.multiple_of((c * half + NS * CNT4 + s) * CHUNK,
                                     CHUNK))

    plsc.subcore_barrier()

    pltpu.sync_copy(den_v, denom_hbm.at[pl.ds(wid * N_NODES, N_NODES)])

    def drain(t, _):
        off = pl.multiple_of(16 * (s + NS * t), 16)
        pltpu.sync_copy(acc.at[pl.ds(off, 16)],
                        numer_hbm.at[c, pl.ds(off, 16)])
        return 0
    lax.fori_loop(0, zcnt, drain, 0)


def _k4(em, att_flat, dst, zeros, e0):
    mesh = plsc.VectorSubcoreMesh(core_axis_name="c", subcore_axis_name="s")
    f = functools.partial(
        pl.kernel,
        out_type=[
            jax.ShapeDtypeStruct((NC, N_NODES, D), jnp.float32),
            jax.ShapeDtypeStruct((NW * N_NODES,), jnp.float32),
        ],
        mesh=mesh,
        scratch_types=[
            pltpu.VMEM((NBUF4, CHUNK), jnp.int32),
            pltpu.VMEM((NBUF4, CHUNK, D), jnp.float32),
            pltpu.VMEM((NBUF4, CHUNK), jnp.float32),
            pltpu.VMEM((N_NODES,), jnp.float32),
            pltpu.VMEM_SHARED((N_NODES, D), jnp.float32),
            pltpu.SemaphoreType.DMA,
            pltpu.SemaphoreType.DMA,
            pltpu.SemaphoreType.DMA,
            pltpu.SemaphoreType.DMA,
        ],
        compiler_params=pltpu.CompilerParams(needs_layout_passes=False),
    )(functools.partial(_k4_body, e0))
    return f(em, att_flat, dst, zeros)


def _k5_body(n0_ref, n1_ref, d0_ref, d1_ref, w1_ref, b1_ref, g_ref, bb_ref,
             w2_ref, b2_ref, out_ref):
    nsum = (n0_ref[0] + n0_ref[1]) + (n1_ref[0] + n1_ref[1])
    ones = jnp.ones((NW, 1), jnp.float32)
    dcol = (_dot(d0_ref[...], ones, ((0,), (0,))) +
            _dot(d1_ref[...], ones, ((0,), (0,))))
    conv = nsum / (dcol + 1e-16)
    h1 = _dot(conv, w1_ref[...], ((1,), (1,))) + b1_ref[...]
    mean = jnp.mean(h1, axis=0, keepdims=True)
    var = jnp.mean((h1 - mean) ** 2, axis=0, keepdims=True)
    h1 = (h1 - mean) / jnp.sqrt(var + 1e-5) * g_ref[...] + bb_ref[...]
    h1 = jnp.tanh(h1)
    out_ref[...] = _dot(h1, w2_ref[...], ((1,), (1,))) + b2_ref[...]


def _k5(n0, n1, d0, d1, out_W1, out_b1, bn_gamma, bn_beta, out_W2, out_b2):
    return pl.pallas_call(
        _k5_body,
        out_shape=jax.ShapeDtypeStruct((N_NODES, D), jnp.float32),
    )(n0, n1, d0.reshape(NW, N_NODES), d1.reshape(NW, N_NODES), out_W1,
      out_b1.reshape(1, D), bn_gamma.reshape(1, D), bn_beta.reshape(1, D),
      out_W2, out_b2.reshape(1, D))


def kernel(x, edge_index, edge_attr, W_init, fn_W1, fn_b1, fn_W2, fn_b2,
           att_vec, out_W1, out_b1, bn_gamma, bn_beta, out_W2, out_b2):
    src = edge_index[0]
    dst = edge_index[1]
    zeros = jnp.zeros((N_NODES, D), jnp.float32)
    h = _k1(x, W_init)
    atts, numers, denoms = [], [], []
    xjs = [_k2(h, src, sl * SLICE_E) for sl in range(NSLICE)]
    for sl in range(NSLICE):
        att3, em = _k3(edge_attr, xjs[sl], fn_W1, fn_b1, fn_W2, fn_b2,
                       att_vec, sl)
        att_s = att3.reshape(SLICE_E)
        n_s, d_s = _k4(em, att_s, dst, zeros, sl * SLICE_E)
        atts.append(att_s)
        numers.append(n_s)
        denoms.append(d_s)
    out = _k5(numers[0], numers[1], denoms[0], denoms[1], out_W1, out_b1,
              bn_gamma, bn_beta, out_W2, out_b2)
    att = jnp.concatenate(atts)
    return out, att


# speedup vs baseline: 14.5693x; 1.4235x over previous
import functools

import jax
import jax.numpy as jnp
from jax import lax
from jax.experimental import pallas as pl
from jax.experimental.pallas import tpu as pltpu
from jax.experimental.pallas import tpu_sc as plsc

N_NODES = 10000
N_EDGES = 320000
D = 128
D_EDGE = 16

NC = 2
NS = 16
NW = NC * NS
CHUNK = 128
NSLICE = 2
SLICE_E = N_EDGES // NSLICE
SLICE_CH = SLICE_E // CHUNK

_PREC = lax.Precision.DEFAULT


def _dot(a, b, dims):
    return lax.dot_general(a, b, (dims, ((), ())), precision=_PREC,
                           preferred_element_type=jnp.float32)


def _k1_body(x_ref, w_ref, h_ref):
    h_ref[...] = _dot(x_ref[...], w_ref[...], ((1,), (1,)))


def _k1(x, w_init):
    return pl.pallas_call(
        _k1_body,
        out_shape=jax.ShapeDtypeStruct((N_NODES, D), jnp.float32),
    )(x, w_init)


CNT2 = SLICE_CH // NW
REM2 = SLICE_CH - CNT2 * NW
NBUF = 3
NGRP = CNT2 // NBUF


def _k2_body(e0, h_hbm, src_hbm, out_hbm, idx_v, row_v,
             sg0, sg1, sg2, ss0, ss1, ss2):
    wid = lax.axis_index("s") * NC + lax.axis_index("c")
    base = wid * (CNT2 * CHUNK)
    semg = [sg0, sg1, sg2]
    sems = [ss0, ss1, ss2]

    pltpu.sync_copy(src_hbm.at[pl.ds(e0 + base, CNT2 * CHUNK)], idx_v)

    def group(g, _):
        ds = []
        for b in range(NBUF):
            off = pl.multiple_of((g * NBUF + b) * CHUNK, CHUNK)
            ds.append(pltpu.async_copy(
                h_hbm.at[idx_v.at[pl.ds(off, CHUNK)]], row_v.at[b], semg[b]))
        ss = []
        for b in range(NBUF):
            off = pl.multiple_of((g * NBUF + b) * CHUNK, CHUNK)
            ds[b].wait()
            ss.append(pltpu.async_copy(
                row_v.at[b], out_hbm.at[pl.ds(base + off, CHUNK)], sems[b]))
        for b in range(NBUF):
            ss[b].wait()
        return 0

    lax.fori_loop(0, NGRP, group, 0)

    @pl.when(wid < REM2)
    def _():
        off = pl.multiple_of((CNT2 * NW + wid) * CHUNK, CHUNK)
        pltpu.sync_copy(src_hbm.at[pl.ds(e0 + off, CHUNK)],
                        idx_v.at[pl.ds(0, CHUNK)])
        pltpu.async_copy(h_hbm.at[idx_v.at[pl.ds(0, CHUNK)]], row_v.at[0],
                         sg0).wait()
        pltpu.sync_copy(row_v.at[0], out_hbm.at[pl.ds(off, CHUNK)])


def _k2(h, src, e0):
    mesh = plsc.VectorSubcoreMesh(core_axis_name="c", subcore_axis_name="s")
    f = functools.partial(
        pl.kernel,
        out_type=jax.ShapeDtypeStruct((SLICE_E, D), jnp.float32),
        mesh=mesh,
        scratch_types=[
            pltpu.VMEM((CNT2 * CHUNK,), jnp.int32),
            pltpu.VMEM((NBUF, CHUNK, D), jnp.float32),
            pltpu.SemaphoreType.DMA,
            pltpu.SemaphoreType.DMA,
            pltpu.SemaphoreType.DMA,
            pltpu.SemaphoreType.DMA,
            pltpu.SemaphoreType.DMA,
            pltpu.SemaphoreType.DMA,
        ],
    )(functools.partial(_k2_body, e0))
    return f(h, src)


EBLK = 3200
NBLK = SLICE_E // EBLK
ABLK = EBLK // 128


def _k3_body(ea_ref, xj_ref, w1_ref, b1_ref, w2_ref, b2_ref, av_ref,
             att_ref, em_ref):
    eat = ea_ref[...]
    nrm = jnp.sqrt(jnp.sum(eat * eat, axis=0, keepdims=True)) + 1e-8
    eant = eat / nrm
    hid = jnp.tanh(_dot(eant, w1_ref[...], ((0,), (1,))) + b1_ref[...])
    ew = _dot(hid, w2_ref[...], ((1,), (1,))) + b2_ref[...]
    m = xj_ref[...] * ew
    att_row = av_ref[...]
    attc = jnp.sum(m * att_row, axis=1, keepdims=True)
    em_ref[...] = m * jnp.exp(attc)
    m3 = m.reshape(ABLK, 128, 128)
    att_ref[...] = jnp.sum(m3 * att_row.reshape(1, 1, 128),
                           axis=2).reshape(1, ABLK, 128)


def _k3(edge_attr, xj, fn_W1, fn_b1, fn_W2, fn_b2, att_vec, sl):
    blk0 = sl * NBLK
    return pl.pallas_call(
        _k3_body,
        grid=(NBLK,),
        in_specs=[
            pl.BlockSpec((D_EDGE, EBLK), lambda i: (0, i + blk0)),
            pl.BlockSpec((EBLK, D), lambda i: (i, 0)),
            pl.BlockSpec((D, D_EDGE), lambda i: (0, 0)),
            pl.BlockSpec((1, D), lambda i: (0, 0)),
            pl.BlockSpec((D, D), lambda i: (0, 0)),
            pl.BlockSpec((1, D), lambda i: (0, 0)),
            pl.BlockSpec((1, D), lambda i: (0, 0)),
        ],
        out_specs=[
            pl.BlockSpec((1, ABLK, 128), lambda i: (i, 0, 0)),
            pl.BlockSpec((EBLK, D), lambda i: (i, 0)),
        ],
        out_shape=[
            jax.ShapeDtypeStruct((NBLK, ABLK, 128), jnp.float32),
            jax.ShapeDtypeStruct((SLICE_E, D), jnp.float32),
        ],
    )(edge_attr.T, xj, fn_W1, fn_b1.reshape(1, D), fn_W2, fn_b2.reshape(1, D),
      att_vec.reshape(1, D))


CNT4 = (SLICE_CH // NC) // NS
REM4 = SLICE_CH // NC - CNT4 * NS
NBUF4 = 2
NGRP4 = CNT4 // NBUF4


def _k4_body(e0, em_hbm, att_hbm, dst_hbm, zeros_hbm, numer_hbm, denom_hbm,
             idx2, em_v, att_v, den_v, acc, sl0, sl1, sc0, sc1):
    c = lax.axis_index("c")
    s = lax.axis_index("s")
    wid = s * NC + c

    def zb(i, _):
        den_v[pl.ds(i * 16, 16)] = jnp.zeros((16,), jnp.float32)
        return 0
    lax.fori_loop(0, N_NODES // 16, zb, 0)

    zoff = pl.multiple_of(s * 624, 8)

    @pl.when(s < NS - 1)
    def _():
        pltpu.sync_copy(zeros_hbm.at[pl.ds(zoff, 624)],
                        acc.at[pl.ds(zoff, 624)])

    @pl.when(s == NS - 1)
    def _():
        off15 = pl.multiple_of(624 * (NS - 1), 8)
        pltpu.sync_copy(zeros_hbm.at[pl.ds(off15, N_NODES - 624 * (NS - 1))],
                        acc.at[pl.ds(off15, N_NODES - 624 * (NS - 1))])
    plsc.subcore_barrier()

    half = SLICE_CH // NC
    start = c * half + s * CNT4
    base = start * CHUNK
    seml = [sl0, sl1]
    semc = [sc0, sc1]

    def denom_chunk(b):
        for t in range(CHUNK // 16):
            iv = idx2[b, pl.ds(t * 16, 16)]
            ev = jnp.exp(att_v[b, pl.ds(t * 16, 16)])
            plsc.addupdate_scatter(den_v, [iv], ev)

    def do_chunk_sync(off):
        pltpu.sync_copy(dst_hbm.at[pl.ds(e0 + off, CHUNK)], idx2.at[0])
        pltpu.sync_copy(em_hbm.at[pl.ds(off, CHUNK)], em_v.at[0])
        pltpu.sync_copy(att_hbm.at[pl.ds(off, CHUNK)], att_v.at[0])
        pltpu.sync_copy(em_v.at[0], acc.at[idx2.at[0]], add=True)
        denom_chunk(0)

    def group(g, _):
        ls = []
        for b in range(NBUF4):
            off = pl.multiple_of(base + (g * NBUF4 + b) * CHUNK, CHUNK)
            ls.append((
                pltpu.async_copy(dst_hbm.at[pl.ds(e0 + off, CHUNK)],
                                 idx2.at[b], seml[b]),
                pltpu.async_copy(em_hbm.at[pl.ds(off, CHUNK)], em_v.at[b],
                                 seml[b]),
                pltpu.async_copy(att_hbm.at[pl.ds(off, CHUNK)], att_v.at[b],
                                 seml[b]),
            ))
        cs = []
        for b in range(NBUF4):
            for d in ls[b]:
                d.wait()
            cs.append(pltpu.async_copy(em_v.at[b], acc.at[idx2.at[b]],
                                       semc[b], add=True))
            denom_chunk(b)
        for d in cs:
            d.wait()
        return 0

    lax.fori_loop(0, NGRP4, group, 0)
    do_chunk_sync(pl.multiple_of(base + NGRP4 * NBUF4 * CHUNK, CHUNK))
    @pl.when(s < REM4)
    def _():
        do_chunk_sync(pl.multiple_of((c * half + NS * CNT4 + s) * CHUNK,
                                     CHUNK))

    plsc.subcore_barrier()

    pltpu.sync_copy(den_v, denom_hbm.at[pl.ds(wid * N_NODES, N_NODES)])

    @pl.when(s < NS - 1)
    def _():
        pltpu.sync_copy(acc.at[pl.ds(zoff, 624)],
                        numer_hbm.at[c, pl.ds(zoff, 624)])

    @pl.when(s == NS - 1)
    def _():
        off15 = pl.multiple_of(624 * (NS - 1), 8)
        pltpu.sync_copy(acc.at[pl.ds(off15, N_NODES - 624 * (NS - 1))],
                        numer_hbm.at[c, pl.ds(off15, N_NODES - 624 * (NS - 1))])


def _k4(em, att_flat, dst, zeros, e0):
    mesh = plsc.VectorSubcoreMesh(core_axis_name="c", subcore_axis_name="s")
    f = functools.partial(
        pl.kernel,
        out_type=[
            jax.ShapeDtypeStruct((NC, N_NODES, D), jnp.float32),
            jax.ShapeDtypeStruct((NW * N_NODES,), jnp.float32),
        ],
        mesh=mesh,
        scratch_types=[
            pltpu.VMEM((NBUF4, CHUNK), jnp.int32),
            pltpu.VMEM((NBUF4, CHUNK, D), jnp.float32),
            pltpu.VMEM((NBUF4, CHUNK), jnp.float32),
            pltpu.VMEM((N_NODES,), jnp.float32),
            pltpu.VMEM_SHARED((N_NODES, D), jnp.float32),
            pltpu.SemaphoreType.DMA,
            pltpu.SemaphoreType.DMA,
            pltpu.SemaphoreType.DMA,
            pltpu.SemaphoreType.DMA,
        ],
        compiler_params=pltpu.CompilerParams(needs_layout_passes=False),
    )(functools.partial(_k4_body, e0))
    return f(em, att_flat, dst, zeros)


def _k5_body(n0_ref, n1_ref, d0_ref, d1_ref, w1_ref, b1_ref, g_ref, bb_ref,
             w2_ref, b2_ref, out_ref):
    nsum = (n0_ref[0] + n0_ref[1]) + (n1_ref[0] + n1_ref[1])
    ones = jnp.ones((NW, 1), jnp.float32)
    dcol = (_dot(d0_ref[...], ones, ((0,), (0,))) +
            _dot(d1_ref[...], ones, ((0,), (0,))))
    conv = nsum / (dcol + 1e-16)
    h1 = _dot(conv, w1_ref[...], ((1,), (1,))) + b1_ref[...]
    mean = jnp.mean(h1, axis=0, keepdims=True)
    var = jnp.mean((h1 - mean) ** 2, axis=0, keepdims=True)
    h1 = (h1 - mean) / jnp.sqrt(var + 1e-5) * g_ref[...] + bb_ref[...]
    h1 = jnp.tanh(h1)
    out_ref[...] = _dot(h1, w2_ref[...], ((1,), (1,))) + b2_ref[...]


def _k5(n0, n1, d0, d1, out_W1, out_b1, bn_gamma, bn_beta, out_W2, out_b2):
    return pl.pallas_call(
        _k5_body,
        out_shape=jax.ShapeDtypeStruct((N_NODES, D), jnp.float32),
    )(n0, n1, d0.reshape(NW, N_NODES), d1.reshape(NW, N_NODES), out_W1,
      out_b1.reshape(1, D), bn_gamma.reshape(1, D), bn_beta.reshape(1, D),
      out_W2, out_b2.reshape(1, D))


def kernel(x, edge_index, edge_attr, W_init, fn_W1, fn_b1, fn_W2, fn_b2,
           att_vec, out_W1, out_b1, bn_gamma, bn_beta, out_W2, out_b2):
    src = edge_index[0]
    dst = edge_index[1]
    zeros = jnp.zeros((N_NODES, D), jnp.float32)
    h = _k1(x, W_init)
    atts, numers, denoms = [], [], []
    xjs = [_k2(h, src, sl * SLICE_E) for sl in range(NSLICE)]
    for sl in range(NSLICE):
        att3, em = _k3(edge_attr, xjs[sl], fn_W1, fn_b1, fn_W2, fn_b2,
                       att_vec, sl)
        att_s = att3.reshape(SLICE_E)
        n_s, d_s = _k4(em, att_s, dst, zeros, sl * SLICE_E)
        atts.append(att_s)
        numers.append(n_s)
        denoms.append(d_s)
    out = _k5(numers[0], numers[1], denoms[0], denoms[1], out_W1, out_b1,
              bn_gamma, bn_beta, out_W2, out_b2)
    att = jnp.concatenate(atts)
    return out, att
